# baseline (speedup 1.0000x reference)
import functools

import jax
import jax.numpy as jnp
from jax import lax
from jax.experimental import pallas as pl
from jax.experimental.pallas import tpu as pltpu
from jax.experimental.pallas import tpu_sc as plsc

_BT = 512
_CH = 128


def _labels_tc(x, center_t):
    B, M, D = x.shape
    K = center_t.shape[2]

    def body(x_ref, ct_ref, lab_ref):
        kio = lax.broadcasted_iota(
            jnp.int32, (_BT, K), 1).astype(jnp.float32)
        cols = []
        for m in range(M):
            xm = x_ref[:, m, :]
            cm = ct_ref[m]
            csq = jnp.sum(cm * cm, axis=0, keepdims=True)
            xsq = jnp.sum(xm * xm, axis=1, keepdims=True)
            dot = lax.dot_general(
                xm, cm, (((1,), (0,)), ((), ())),
                preferred_element_type=jnp.float32)
            dist = (csq - 2.0 * dot) + xsq
            neg = -jnp.sqrt(dist)
            mx = jnp.max(neg, axis=1, keepdims=True)
            sel = jnp.where(neg == mx, kio, float(K - 1))
            cols.append(jnp.min(sel, axis=1)[:, None])
        lab_ref[...] = jnp.concatenate(cols, axis=1).astype(jnp.int32)

    return pl.pallas_call(
        body,
        grid=(B // _BT,),
        in_specs=[
            pl.BlockSpec((_BT, M, D), lambda j: (j, 0, 0)),
            pl.BlockSpec((M, D, K), lambda j: (0, 0, 0)),
        ],
        out_specs=pl.BlockSpec((_BT, M), lambda j: (j, 0)),
        out_shape=jax.ShapeDtypeStruct((B, M), jnp.int32),
    )(x, center_t)


def _gather_sc(table, gidx3):
    NW, NCH, CH = gidx3.shape
    D = table.shape[1]
    info = plsc.get_sparse_core_info()
    NC = info.num_cores
    mesh = plsc.VectorSubcoreMesh(core_axis_name="c", subcore_axis_name="s")

    @functools.partial(
        pl.kernel,
        out_type=jax.ShapeDtypeStruct((NW * NCH * CH, D), jnp.float32),
        mesh=mesh,
        scratch_types=[
            pltpu.VMEM((NCH, CH), jnp.int32),
            pltpu.VMEM((CH, D), jnp.float32),
            pltpu.VMEM((CH, D), jnp.float32),
            pltpu.SemaphoreType.DMA,
            pltpu.SemaphoreType.DMA,
        ],
    )
    def gk(idx_hbm, table_hbm, out_hbm, idx_v, buf0, buf1, sem0, sem1):
        wid = lax.axis_index("s") * NC + lax.axis_index("c")
        base = wid * (NCH * CH)
        pltpu.sync_copy(idx_hbm.at[wid], idx_v)
        bufs = (buf0, buf1)
        sems = (sem0, sem1)
        cps = [pltpu.async_copy(table_hbm.at[idx_v.at[0]], buf0, sem0), None]
        for c in range(NCH):
            cur = c % 2
            nxt = (c + 1) % 2
            if c + 1 < NCH:
                cps[nxt] = pltpu.async_copy(
                    table_hbm.at[idx_v.at[c + 1]], bufs[nxt], sems[nxt])
            cps[cur].wait()
            pltpu.sync_copy(bufs[cur], out_hbm.at[pl.ds(base + c * CH, CH)])

    return gk(gidx3, table)


def kernel(x, center):
    B, M, D = x.shape
    K = center.shape[1]
    center_t = jnp.transpose(center, (0, 2, 1))
    table = center.reshape(M * K, D)
    info = plsc.get_sparse_core_info()
    NW = info.num_cores * info.num_subcores
    offs = (jnp.arange(M, dtype=jnp.int32) * K)[None, :]

    H = B // 2
    NCH = (H * M) // (NW * _CH)
    halves = []
    labs = []
    for h in range(2):
        lab = _labels_tc(x[h * H:(h + 1) * H], center_t)
        labs.append(lab)
        gidx3 = (lab + offs).reshape(NW, NCH, _CH)
        halves.append(_gather_sc(table, gidx3).reshape(H, M, D))
    out = jnp.concatenate(halves, axis=0)
    labels = jnp.concatenate(labs, axis=0)
    return (out, center, labels[..., None])


# speedup vs baseline: 1.2888x; 1.2888x over previous
import functools

import jax
import jax.numpy as jnp
from jax import lax
from jax.experimental import pallas as pl
from jax.experimental.pallas import tpu as pltpu
from jax.experimental.pallas import tpu_sc as plsc

_BT = 1024
_CH = 128


def _labels_tc(x, center_t):
    B, M, D = x.shape
    K = center_t.shape[2]

    def body(x_ref, ct_ref, lab_ref):
        kio = lax.broadcasted_iota(
            jnp.int32, (_BT, K), 1).astype(jnp.float32)
        cols = []
        for m in range(M):
            xm = x_ref[:, m, :]
            cm = ct_ref[m]
            csq = jnp.sum(cm * cm, axis=0, keepdims=True)
            xsq = jnp.sum(xm * xm, axis=1, keepdims=True)
            dot = lax.dot_general(
                xm, cm, (((1,), (0,)), ((), ())),
                preferred_element_type=jnp.float32)
            dist = (csq - 2.0 * dot) + xsq
            neg = -jnp.sqrt(dist)
            mx = jnp.max(neg, axis=1, keepdims=True)
            sel = jnp.where(neg == mx, kio, float(K - 1))
            cols.append(jnp.min(sel, axis=1)[:, None])
        lab_ref[...] = jnp.concatenate(cols, axis=1).astype(jnp.int32)

    return pl.pallas_call(
        body,
        grid=(B // _BT,),
        in_specs=[
            pl.BlockSpec((_BT, M, D), lambda j: (j, 0, 0)),
            pl.BlockSpec((M, D, K), lambda j: (0, 0, 0)),
        ],
        out_specs=pl.BlockSpec((_BT, M), lambda j: (j, 0)),
        out_shape=jax.ShapeDtypeStruct((B, M), jnp.int32),
    )(x, center_t)


def _gather_sc(table, gidx3):
    NW, NCH, CH = gidx3.shape
    D = table.shape[1]
    info = plsc.get_sparse_core_info()
    NC = info.num_cores
    mesh = plsc.VectorSubcoreMesh(core_axis_name="c", subcore_axis_name="s")

    @functools.partial(
        pl.kernel,
        out_type=jax.ShapeDtypeStruct((NW * NCH * CH, D), jnp.float32),
        mesh=mesh,
        scratch_types=[
            pltpu.VMEM((NCH, CH), jnp.int32),
            pltpu.VMEM((CH, D), jnp.float32),
            pltpu.VMEM((CH, D), jnp.float32),
            pltpu.SemaphoreType.DMA,
            pltpu.SemaphoreType.DMA,
        ],
    )
    def gk(idx_hbm, table_hbm, out_hbm, idx_v, buf0, buf1, sem0, sem1):
        wid = lax.axis_index("s") * NC + lax.axis_index("c")
        base = wid * (NCH * CH)
        pltpu.sync_copy(idx_hbm.at[wid], idx_v)
        bufs = (buf0, buf1)
        sems = (sem0, sem1)
        cps = [pltpu.async_copy(table_hbm.at[idx_v.at[0]], buf0, sem0), None]
        for c in range(NCH):
            cur = c % 2
            nxt = (c + 1) % 2
            if c + 1 < NCH:
                cps[nxt] = pltpu.async_copy(
                    table_hbm.at[idx_v.at[c + 1]], bufs[nxt], sems[nxt])
            cps[cur].wait()
            pltpu.sync_copy(bufs[cur], out_hbm.at[pl.ds(base + c * CH, CH)])

    return gk(gidx3, table)


def kernel(x, center):
    B, M, D = x.shape
    K = center.shape[1]
    center_t = jnp.transpose(center, (0, 2, 1))
    labels = _labels_tc(x, center_t)

    info = plsc.get_sparse_core_info()
    NW = info.num_cores * info.num_subcores
    NCH = (B * M) // (NW * _CH)
    gidx = labels + (jnp.arange(M, dtype=jnp.int32) * K)[None, :]
    gidx3 = gidx.reshape(NW, NCH, _CH)
    out = _gather_sc(center.reshape(M * K, D), gidx3).reshape(B, M, D)
    return (out, center, labels[..., None])
